# baseline (speedup 1.0000x reference)
import jax
import jax.numpy as jnp
from jax.experimental import pallas as pl
from jax.experimental.pallas import tpu as pltpu

B, H, W, C = 64, 32, 32, 384
HD = 64
BC = 4
NB = B // BC
M = BC * H * W
N_STAT = float(B * H * W)
EPS = 1e-5
F32 = jnp.float32


def _matmul2d(x, w):
    return jax.lax.dot_general(x, w, (((1,), (0,)), ((), ())),
                               preferred_element_type=F32)


def _store_shifted(s_ref, x):
    bc = x.shape[0]
    cin = x.shape[3]
    s_ref[:, 1:H + 1, 0:1, 0:cin] = jnp.zeros((bc, H, 1, cin), F32)
    s_ref[:, 1:H + 1, 1:W, 0:cin] = x[:, :, 0:W - 1, :]
    s_ref[:, 1:H + 1, :, cin:2 * cin] = x
    s_ref[:, 1:H + 1, 0:W - 1, 2 * cin:3 * cin] = x[:, :, 1:W, :]
    s_ref[:, 1:H + 1, W - 1:W, 2 * cin:3 * cin] = jnp.zeros((bc, H, 1, cin), F32)
    return s_ref[:, :, :, 0:3 * cin].reshape(bc * (H + 2) * W, 3 * cin)


def _conv3x3_small(s_ref, x, wj, bias, cout):
    sall = _store_shifted(s_ref, x)
    p = _matmul2d(sall, wj).reshape(x.shape[0], H + 2, W, 3 * cout)
    return (p[:, 0:H, :, 0:cout] + p[:, 1:H + 1, :, cout:2 * cout]
            + p[:, 2:H + 2, :, 2 * cout:3 * cout] + bias)


def _conv3x3_64(s_ref, x, w01, w2, bias):
    sall = _store_shifted(s_ref, x)
    p01 = _matmul2d(sall, w01).reshape(x.shape[0], H + 2, W, 128)
    p2 = _matmul2d(sall, w2).reshape(x.shape[0], H + 2, W, 64)
    return (p01[:, 0:H, :, 0:64] + p01[:, 1:H + 1, :, 64:128]
            + p2[:, 2:H + 2, :, :] + bias)


def _pack_pairs(y1s_ref, pair_base, y):
    hw = H * W
    npair = y.shape[0] // (2 * hw)
    for pair in range(npair):
        base = (pair_base + pair) * hw
        y1s_ref[pl.ds(base, hw), 0:HD] = y[2 * pair * hw:(2 * pair + 1) * hw]
        y1s_ref[pl.ds(base, hw), HD:2 * HD] = y[(2 * pair + 1) * hw:
                                                (2 * pair + 2) * hw]


def _unpack_pairs(y1s_ref, pair_base, npair, a, c):
    hw = H * W
    ap = jnp.concatenate([a, a], axis=1)
    cp = jnp.concatenate([c, c], axis=1)
    v = jnp.maximum(y1s_ref[pl.ds(pair_base * hw, npair * hw), :]
                    * ap + cp, 0.0)
    parts = []
    for pair in range(npair):
        blkv = v[pair * hw:(pair + 1) * hw, :]
        parts.append(blkv[:, 0:HD].reshape(1, H, W, HD))
        parts.append(blkv[:, HD:2 * HD].reshape(1, H, W, HD))
    return jnp.concatenate(parts, axis=0)


def _bn_affine(st, row, g, be, width):
    mean = st[row:row + 1, 0:width] / N_STAT
    var = st[row + 1:row + 2, 0:width] / N_STAT - mean * mean
    a = g * jax.lax.rsqrt(var + EPS)
    return a, be - mean * a


def _mega_body(x_ref, w1_ref, b1_ref, g1_ref, be1_ref, w2_ref, b2_ref,
               w3_ref, b3_ref, w4a_ref, w4b_ref, b4_ref, g4_ref, be4_ref,
               w5_ref, b5_ref, g5_ref, be5_ref, w6a_ref, w6b_ref, b6_ref,
               w7_ref, b7_ref, out_ref, y1s_ref, y5p_ref, s_ref, st_ref):
    p = pl.program_id(0)
    i = pl.program_id(1)
    ii = i // 2
    mc = 2 * M

    @pl.when((p == 0) & (i == 0))
    def _init():
        s_ref[...] = jnp.zeros(s_ref.shape, F32)
        st_ref[...] = jnp.zeros(st_ref.shape, F32)

    @pl.when(p == 0)
    def _phase0():
        y = _matmul2d(x_ref[...].reshape(M, C), w1_ref[...]) + b1_ref[...]
        _pack_pairs(y1s_ref, i * (BC // 2), y)
        st_ref[0:1, 0:HD] += jnp.sum(y, axis=0).reshape(1, HD)
        st_ref[1:2, 0:HD] += jnp.sum(y * y, axis=0).reshape(1, HD)

    @pl.when((p == 1) & (i % 2 == 0))
    def _phase1():
        a1, c1 = _bn_affine(st_ref[...], 0, g1_ref[...], be1_ref[...], HD)
        x1 = _unpack_pairs(y1s_ref, ii * BC, BC, a1, c1)
        t = jnp.maximum(
            _conv3x3_small(s_ref, x1, w2_ref[...], b2_ref[...], HD // 4), 0.0)
        logit = _matmul2d(t.reshape(mc, HD // 4), w3_ref[...]) + b3_ref[...]
        attn = jax.nn.sigmoid(logit).reshape(2 * BC, H, W, HD)
        y4 = _conv3x3_64(s_ref, x1 * attn, w4a_ref[...], w4b_ref[...],
                         b4_ref[...])
        _pack_pairs(y1s_ref, ii * BC, y4.reshape(mc, HD))
        st_ref[2:3, 0:HD] += jnp.sum(y4, axis=(0, 1, 2)).reshape(1, HD)
        st_ref[3:4, 0:HD] += jnp.sum(y4 * y4, axis=(0, 1, 2)).reshape(1, HD)

    @pl.when((p == 2) & (i % 2 == 0))
    def _phase2():
        a4, c4 = _bn_affine(st_ref[...], 2, g4_ref[...], be4_ref[...], HD)
        x4 = _unpack_pairs(y1s_ref, ii * BC, BC, a4, c4)
        y5 = _conv3x3_small(s_ref, x4, w5_ref[...], b5_ref[...], HD // 2)
        for j in range(2 * BC):
            y5p_ref[2 * ii + j // 4, :, :, 32 * (j % 4):32 * (j % 4 + 1)] = y5[j]
        st_ref[4:5, 0:HD // 2] += jnp.sum(y5, axis=(0, 1, 2)).reshape(1, HD // 2)
        st_ref[5:6, 0:HD // 2] += jnp.sum(y5 * y5, axis=(0, 1, 2)).reshape(1, HD // 2)

    @pl.when((p == 3) & (i % 2 == 0))
    def _phase3():
        a5, c5 = _bn_affine(st_ref[...], 4, g5_ref[...], be5_ref[...], HD // 2)
        a5p = jnp.concatenate([a5] * 4, axis=1)
        c5p = jnp.concatenate([c5] * 4, axis=1)
        x5p0 = jnp.maximum(y5p_ref[2 * ii] * a5p + c5p, 0.0)
        x5p1 = jnp.maximum(y5p_ref[2 * ii + 1] * a5p + c5p, 0.0)
        x5 = jnp.stack(
            [x5p0[:, :, 32 * j:32 * (j + 1)] for j in range(4)]
            + [x5p1[:, :, 32 * j:32 * (j + 1)] for j in range(4)], axis=0)
        t = jnp.maximum(
            _conv3x3_64(s_ref, x5, w6a_ref[...], w6b_ref[...], b6_ref[...]),
            0.0)
        logit = _matmul2d(t.reshape(mc, 64), w7_ref[...]) + b7_ref[...]
        out_ref[...] = jax.nn.sigmoid(logit.reshape(mc // 128, 128))


def _wconst(shape):
    nd = len(shape)
    return pl.BlockSpec(shape, lambda p, i, _n=nd: (0,) * _n)


def kernel(dino_features, W1, b1, g1, be1, W2, b2, W3, b3, W4, b4, g4, be4,
           W5, b5, g5, be5, W6, b6, W7, b7):
    w1 = W1.reshape(HD, C).T
    wt2 = W2.transpose(2, 3, 1, 0).reshape(3, 3 * HD, HD // 4)
    w2 = jnp.concatenate([wt2[0], wt2[1], wt2[2]], axis=1)
    w3 = jnp.tile(W3.reshape(1, HD // 4).T, (1, HD))
    wt4 = W4.transpose(2, 3, 1, 0).reshape(3, 3 * HD, HD)
    w4a = jnp.concatenate([wt4[0], wt4[1]], axis=1)
    w4b = wt4[2]
    wt5 = W5.transpose(2, 3, 1, 0).reshape(3, 3 * HD, HD // 2)
    w5 = jnp.concatenate([wt5[0], wt5[1], wt5[2]], axis=1)
    wt6 = W6.transpose(2, 3, 1, 0).reshape(3, 3 * (HD // 2), 64)
    w6a = jnp.concatenate([wt6[0], wt6[1]], axis=1)
    w6b = wt6[2]
    w7 = W7.reshape(1, 64).T

    x_spec = pl.BlockSpec(
        (BC, H, W, C),
        lambda p, i: (jnp.where(p == 0, i, NB - 1), 0, 0, 0))
    out_spec = pl.BlockSpec(
        (2 * M // 128, 128),
        lambda p, i: (jnp.where(p == 3, i // 2, 0), 0))

    out2d = pl.pallas_call(
        _mega_body,
        grid=(4, NB),
        in_specs=[
            x_spec,
            _wconst((C, HD)), _wconst((1, HD)), _wconst((1, HD)),
            _wconst((1, HD)),
            _wconst((3 * HD, 3 * (HD // 4))), _wconst((1, HD // 4)),
            _wconst((HD // 4, HD)), _wconst((1, 1)),
            _wconst((3 * HD, 2 * HD)), _wconst((3 * HD, HD)),
            _wconst((1, HD)), _wconst((1, HD)), _wconst((1, HD)),
            _wconst((3 * HD, 3 * (HD // 2))), _wconst((1, HD // 2)),
            _wconst((1, HD // 2)), _wconst((1, HD // 2)),
            _wconst((3 * (HD // 2), 128)), _wconst((3 * (HD // 2), 64)),
            _wconst((1, 64)),
            _wconst((64, 1)), _wconst((1, 1)),
        ],
        out_specs=out_spec,
        out_shape=jax.ShapeDtypeStruct((B * H * W // 128, 128), F32),
        scratch_shapes=[
            pltpu.VMEM((B * H * W // 2, 128), F32),
            pltpu.VMEM((NB, H, W, 128), F32),
            pltpu.VMEM((2 * BC, H + 2, W, 3 * HD), F32),
            pltpu.VMEM((8, 128), F32),
        ],
    )(dino_features, w1, b1.reshape(1, HD), g1.reshape(1, HD),
      be1.reshape(1, HD), w2, b2.reshape(1, HD // 4), w3, b3.reshape(1, 1),
      w4a, w4b, b4.reshape(1, HD), g4.reshape(1, HD), be4.reshape(1, HD),
      w5, b5.reshape(1, HD // 2), g5.reshape(1, HD // 2),
      be5.reshape(1, HD // 2), w6a, w6b, b6.reshape(1, 64), w7,
      b7.reshape(1, 1))
    return out2d.reshape(B, H, W, 1)
